# baseline (speedup 1.0000x reference)
import functools

import jax
import jax.numpy as jnp
from jax import lax
from jax.experimental import pallas as pl
from jax.experimental.pallas import tpu as pltpu
from jax.experimental.pallas import tpu_sc as plsc

B, H, W = 4, 512, 512
K = 1024
L = 16
NTILES = 16
IMGS_PER_CORE = B // 2
ROWS = H // NTILES
CHUNK = ROWS * W
KROWS = K // L


def _sc_body(ndvi_hbm, slic_hbm, out_hbm,
             ndvi_a, slic_a, ndvi_b, slic_b, o0_v, o1_v, o2_v,
             hist_v, rowidx_v, sh_hist, sem_in, sem_out):
    c = lax.axis_index("c")
    s = lax.axis_index("s")
    ones_f = jnp.ones((L,), jnp.float32)
    zeros_f = jnp.zeros((L,), jnp.float32)
    iota = lax.iota(jnp.int32, L)
    r0 = s * ROWS

    in_bufs = ((ndvi_a, slic_a), (ndvi_b, slic_b))
    in_handles = []
    for bb in range(IMGS_PER_CORE):
        b = c * IMGS_PER_CORE + bb
        nv_ref, sv_ref = in_bufs[bb]
        in_handles.append((
            pltpu.async_copy(
                ndvi_hbm.at[b, pl.ds(r0, ROWS), :], nv_ref, sem_in),
            pltpu.async_copy(
                slic_hbm.at[b, pl.ds(r0, ROWS), :], sv_ref, sem_in),
        ))

    for j in range(KROWS // L):
        rowidx_v[pl.ds(j * L, L)] = iota + j * L

    @plsc.parallel_loop(0, CHUNK, step=L, unroll=8)
    def _(i):
        o2_v[i >> 9, pl.ds(i & (W - 1), L)] = zeros_f

    out_handles = ()
    for bb in range(IMGS_PER_CORE):
        b = c * IMGS_PER_CORE + bb
        nv_ref, sv_ref = in_bufs[bb]

        def zero_body(j, _):
            hist_v[j, :] = zeros_f
            return 0
        lax.fori_loop(0, KROWS, zero_body, 0, unroll=8)

        @pl.when(s == 0)
        def _():
            pltpu.sync_copy(hist_v, sh_hist)

        hn, hs = in_handles[bb]
        hn.wait()
        hs.wait()

        @plsc.parallel_loop(0, CHUNK, step=L, unroll=16)
        def _(i):
            r = i >> 9
            cc = i & (W - 1)
            nv = nv_ref[r, pl.ds(cc, L)]
            sv = sv_ref[r, pl.ds(cc, L)]
            m = nv > 0.0
            plsc.addupdate_scatter(
                hist_v, [sv >> 4, sv & 15], ones_f, mask=m)
            sid = jnp.where(m, sv, 0)
            sv_ref[r, pl.ds(cc, L)] = sid | jnp.where(m, 1 << 10, 0)

        plsc.subcore_barrier()
        pltpu.sync_copy(hist_v, sh_hist.at[rowidx_v], add=True)
        plsc.subcore_barrier()
        pltpu.sync_copy(sh_hist, hist_v)
        plsc.subcore_barrier()

        for h in out_handles:
            h.wait()

        @plsc.parallel_loop(0, CHUNK, step=L, unroll=16)
        def _(i):
            r = i >> 9
            cc = i & (W - 1)
            enc = sv_ref[r, pl.ds(cc, L)]
            sid = enc & (K - 1)
            cnt1 = plsc.load_gather(hist_v, [(enc >> 4) & 63, enc & 15])
            apply = (cnt1 > 0.0) & (sid > 0)
            one = jnp.where(apply | (enc > (K - 1)), 1.0, 0.0)
            o1_v[r, pl.ds(cc, L)] = one
            o0_v[r, pl.ds(cc, L)] = 1.0 - one

        out_handles = (
            pltpu.async_copy(o0_v, out_hbm.at[b, 0, pl.ds(r0, ROWS), :], sem_out),
            pltpu.async_copy(o1_v, out_hbm.at[b, 1, pl.ds(r0, ROWS), :], sem_out),
            pltpu.async_copy(o2_v, out_hbm.at[b, 2, pl.ds(r0, ROWS), :], sem_out),
        )

    for h in out_handles:
        h.wait()


@jax.jit
def _run(ndvi, slic):
    mesh = plsc.VectorSubcoreMesh(core_axis_name="c", subcore_axis_name="s")
    fn = functools.partial(
        pl.kernel,
        mesh=mesh,
        compiler_params=pltpu.CompilerParams(needs_layout_passes=False),
        out_type=jax.ShapeDtypeStruct((B, 3, H, W), jnp.float32),
        scratch_types=[
            pltpu.VMEM((ROWS, W), jnp.float32),
            pltpu.VMEM((ROWS, W), jnp.int32),
            pltpu.VMEM((ROWS, W), jnp.float32),
            pltpu.VMEM((ROWS, W), jnp.int32),
            pltpu.VMEM((ROWS, W), jnp.float32),
            pltpu.VMEM((ROWS, W), jnp.float32),
            pltpu.VMEM((ROWS, W), jnp.float32),
            pltpu.VMEM((KROWS, L), jnp.float32),
            pltpu.VMEM((KROWS,), jnp.int32),
            pltpu.VMEM_SHARED((KROWS, L), jnp.float32),
            pltpu.SemaphoreType.DMA,
            pltpu.SemaphoreType.DMA,
        ],
    )(_sc_body)
    return fn(ndvi, slic)


def kernel(image, ndvi, slic):
    del image
    return _run(ndvi, slic)


# speedup vs baseline: 1.0186x; 1.0186x over previous
import functools

import jax
import jax.numpy as jnp
from jax import lax
from jax.experimental import pallas as pl
from jax.experimental.pallas import tpu as pltpu
from jax.experimental.pallas import tpu_sc as plsc

B, H, W = 4, 512, 512
K = 1024
L = 16
NTILES = 16
IMGS_PER_CORE = B // 2
ROWS = H // NTILES
CHUNK = ROWS * W
KROWS = K // L


def _sc_body(ndvi_hbm, slic_hbm, out_hbm,
             ndvi_v, slic_a, slic_b, o0_v, o1_v, o2_v,
             hist1_v, hist2_v, rowidx_v, sh, sem_in, sem_out):
    c = lax.axis_index("c")
    s = lax.axis_index("s")
    ones_f = jnp.ones((L,), jnp.float32)
    zeros_f = jnp.zeros((L,), jnp.float32)
    zeros_i = jnp.zeros((L,), jnp.int32)
    iota = lax.iota(jnp.int32, L)
    lane0 = iota < 1
    r0 = s * ROWS
    b_a = c * IMGS_PER_CORE
    b_b = b_a + 1

    h_in = (
        pltpu.async_copy(
            ndvi_hbm.at[b_a, pl.ds(r0, ROWS), :], ndvi_v, sem_in),
        pltpu.async_copy(
            slic_hbm.at[b_a, pl.ds(r0, ROWS), :], slic_a, sem_in),
        pltpu.async_copy(
            slic_hbm.at[b_b, pl.ds(r0, ROWS), :], slic_b, sem_in),
    )

    for j in range(KROWS // L):
        rowidx_v[pl.ds(j * L, L)] = iota + j * L

    def zero_body(j, _):
        hist1_v[j, :] = zeros_f
        hist2_v[j, :] = zeros_f
        return 0
    lax.fori_loop(0, KROWS, zero_body, 0, unroll=8)

    @pl.when(s == 0)
    def _():
        pltpu.sync_copy(hist2_v, sh)

    @plsc.parallel_loop(0, CHUNK, step=L, unroll=8)
    def _(i):
        o2_v[i >> 9, pl.ds(i & (W - 1), L)] = zeros_f

    h_o2 = (
        pltpu.async_copy(o2_v, out_hbm.at[b_a, 2, pl.ds(r0, ROWS), :], sem_out),
        pltpu.async_copy(o2_v, out_hbm.at[b_b, 2, pl.ds(r0, ROWS), :], sem_out),
    )

    def p1_step(i, nv_ref, sv_ref, hist_v):
        r = i >> 9
        cc = i & (W - 1)
        nv = nv_ref[r, pl.ds(cc, L)]
        sv = sv_ref[r, pl.ds(cc, L)]
        m = nv > 0.0
        plsc.addupdate_scatter(hist_v, [sv >> 4, sv & 15], ones_f, mask=m)
        sv_ref[r, pl.ds(cc, L)] = jnp.where(m, sv + K, 0)

    def p2_step(i, sv_ref, hist_v):
        r = i >> 9
        cc = i & (W - 1)
        enc = sv_ref[r, pl.ds(cc, L)]
        cnt1 = plsc.load_gather(hist_v, [(enc >> 4) & (KROWS - 1), enc & 15])
        one = jnp.where((cnt1 > 0.0) | (enc > (K - 1)), 1.0, 0.0)
        o1_v[r, pl.ds(cc, L)] = one
        o0_v[r, pl.ds(cc, L)] = 1.0 - one

    def combine(hist_v, rezero, post_barrier=None):
        plsc.subcore_barrier()
        handle = post_barrier() if post_barrier is not None else None
        pltpu.sync_copy(hist_v, sh.at[rowidx_v], add=True)
        plsc.subcore_barrier()
        pltpu.sync_copy(sh, hist_v)
        plsc.store_scatter(hist_v, [zeros_i, zeros_i], zeros_f, mask=lane0)
        if rezero:
            plsc.subcore_barrier()

            @pl.when(s == 0)
            def _():
                pltpu.sync_copy(hist2_v, sh)
        return handle

    for h in h_in:
        h.wait()

    @plsc.parallel_loop(0, CHUNK, step=L, unroll=16)
    def _(i):
        p1_step(i, ndvi_v, slic_a, hist1_v)

    def start_ndvi_b():
        return pltpu.async_copy(
            ndvi_hbm.at[b_b, pl.ds(r0, ROWS), :], ndvi_v, sem_in)

    h_nb = combine(hist1_v, rezero=True, post_barrier=start_ndvi_b)

    h_nb.wait()

    @plsc.parallel_loop(0, CHUNK, step=L, unroll=16)
    def _(i):
        p2_step(i, slic_a, hist1_v)
        p1_step(i, ndvi_v, slic_b, hist2_v)

    h_out_a = (
        pltpu.async_copy(o0_v, out_hbm.at[b_a, 0, pl.ds(r0, ROWS), :], sem_out),
        pltpu.async_copy(o1_v, out_hbm.at[b_a, 1, pl.ds(r0, ROWS), :], sem_out),
    )

    combine(hist2_v, rezero=False)

    for h in h_out_a:
        h.wait()

    @plsc.parallel_loop(0, CHUNK, step=L, unroll=16)
    def _(i):
        p2_step(i, slic_b, hist2_v)

    h_out_b = (
        pltpu.async_copy(o0_v, out_hbm.at[b_b, 0, pl.ds(r0, ROWS), :], sem_out),
        pltpu.async_copy(o1_v, out_hbm.at[b_b, 1, pl.ds(r0, ROWS), :], sem_out),
    )
    for h in h_o2 + h_out_b:
        h.wait()


@jax.jit
def _run(ndvi, slic):
    mesh = plsc.VectorSubcoreMesh(core_axis_name="c", subcore_axis_name="s")
    fn = functools.partial(
        pl.kernel,
        mesh=mesh,
        compiler_params=pltpu.CompilerParams(needs_layout_passes=False),
        out_type=jax.ShapeDtypeStruct((B, 3, H, W), jnp.float32),
        scratch_types=[
            pltpu.VMEM((ROWS, W), jnp.float32),
            pltpu.VMEM((ROWS, W), jnp.int32),
            pltpu.VMEM((ROWS, W), jnp.int32),
            pltpu.VMEM((ROWS, W), jnp.float32),
            pltpu.VMEM((ROWS, W), jnp.float32),
            pltpu.VMEM((ROWS, W), jnp.float32),
            pltpu.VMEM((KROWS, L), jnp.float32),
            pltpu.VMEM((KROWS, L), jnp.float32),
            pltpu.VMEM((KROWS,), jnp.int32),
            pltpu.VMEM_SHARED((KROWS, L), jnp.float32),
            pltpu.SemaphoreType.DMA,
            pltpu.SemaphoreType.DMA,
        ],
    )(_sc_body)
    return fn(ndvi, slic)


def kernel(image, ndvi, slic):
    del image
    return _run(ndvi, slic)
